# baseline (speedup 1.0000x reference)
import functools

import jax
import jax.numpy as jnp
from jax import lax
from jax.experimental import pallas as pl
from jax.experimental.pallas import tpu as pltpu

N_NODES = 10000
D_IN = 128
D_H = 64
HEADS = 8
N_OUT = 8
ROW_BLK = 1000


def _prologue1_body(v_ref, fw_ref, fb_ref, W_ref, Up_ref, cv_ref, Ws_ref,
                    y_ref, pq_ref, self_ref):
    h = v_ref[...] @ fw_ref[...] + fb_ref[...]
    h = jnp.where(h > 0, h, jnp.exp(h) - 1.0)
    y_ref[...] = h @ W_ref[...]
    pq_ref[...] = h @ Up_ref[...] - cv_ref[...]
    self_ref[...] = h @ Ws_ref[...]


def _prologue1(verts, fc0_w, fc0_b, W, Upad, cvec, Wself):
    grid = (N_NODES // ROW_BLK,)
    return pl.pallas_call(
        _prologue1_body,
        grid=grid,
        in_specs=[
            pl.BlockSpec((ROW_BLK, D_IN), lambda i: (i, 0)),
            pl.BlockSpec((D_IN, D_H), lambda i: (0, 0)),
            pl.BlockSpec((D_H,), lambda i: (0,)),
            pl.BlockSpec((D_H, HEADS * D_H), lambda i: (0, 0)),
            pl.BlockSpec((D_H, 16), lambda i: (0, 0)),
            pl.BlockSpec((16,), lambda i: (0,)),
            pl.BlockSpec((D_H, D_H), lambda i: (0, 0)),
        ],
        out_specs=[
            pl.BlockSpec((ROW_BLK, HEADS * D_H), lambda i: (i, 0)),
            pl.BlockSpec((ROW_BLK, 16), lambda i: (i, 0)),
            pl.BlockSpec((ROW_BLK, D_H), lambda i: (i, 0)),
        ],
        out_shape=[
            jax.ShapeDtypeStruct((N_NODES, HEADS * D_H), jnp.float32),
            jax.ShapeDtypeStruct((N_NODES, 16), jnp.float32),
            jax.ShapeDtypeStruct((N_NODES, D_H), jnp.float32),
        ],
    )(verts, fc0_w, fc0_b, W, Upad, cvec, Wself)


def _mid_body(ad_ref, self_ref, b_ref, W_ref, Up_ref, cv_ref, Ws_ref,
              y_ref, pq_ref, self2_ref):
    agg = ad_ref[0, :, :D_H] + ad_ref[1, :, :D_H]
    deg = ad_ref[0, :, D_H:D_H + 1] + ad_ref[1, :, D_H:D_H + 1]
    x = (agg + self_ref[...]) / (deg + 1.0) + b_ref[...]
    x = jnp.where(x > 0, x, jnp.exp(x) - 1.0)
    y_ref[...] = x @ W_ref[...]
    pq_ref[...] = x @ Up_ref[...] - cv_ref[...]
    self2_ref[...] = x @ Ws_ref[...]


def _mid(aggdeg, self_msg, b, W, Upad, cvec, Wself):
    grid = (N_NODES // ROW_BLK,)
    return pl.pallas_call(
        _mid_body,
        grid=grid,
        in_specs=[
            pl.BlockSpec((2, ROW_BLK, 80), lambda i: (0, i, 0)),
            pl.BlockSpec((ROW_BLK, D_H), lambda i: (i, 0)),
            pl.BlockSpec((D_H,), lambda i: (0,)),
            pl.BlockSpec((D_H, HEADS * D_H), lambda i: (0, 0)),
            pl.BlockSpec((D_H, 16), lambda i: (0, 0)),
            pl.BlockSpec((16,), lambda i: (0,)),
            pl.BlockSpec((D_H, D_H), lambda i: (0, 0)),
        ],
        out_specs=[
            pl.BlockSpec((ROW_BLK, HEADS * D_H), lambda i: (i, 0)),
            pl.BlockSpec((ROW_BLK, 16), lambda i: (i, 0)),
            pl.BlockSpec((ROW_BLK, D_H), lambda i: (i, 0)),
        ],
        out_shape=[
            jax.ShapeDtypeStruct((N_NODES, HEADS * D_H), jnp.float32),
            jax.ShapeDtypeStruct((N_NODES, 16), jnp.float32),
            jax.ShapeDtypeStruct((N_NODES, D_H), jnp.float32),
        ],
    )(aggdeg, self_msg, b, W, Upad, cvec, Wself)


def _final_body(ad_ref, self_ref, b_ref, lw_ref, lb_ref, out_ref):
    agg = ad_ref[0, :, :D_H] + ad_ref[1, :, :D_H]
    deg = ad_ref[0, :, D_H:D_H + 1] + ad_ref[1, :, D_H:D_H + 1]
    x = (agg + self_ref[...]) / (deg + 1.0) + b_ref[...]
    x = jnp.where(x > 0, x, jnp.exp(x) - 1.0)
    out_ref[...] = jnp.tanh(x @ lw_ref[...] + lb_ref[...])


def _final(aggdeg, self_msg, b, lin_w, lin_b):
    grid = (N_NODES // ROW_BLK,)
    return pl.pallas_call(
        _final_body,
        grid=grid,
        in_specs=[
            pl.BlockSpec((2, ROW_BLK, 80), lambda i: (0, i, 0)),
            pl.BlockSpec((ROW_BLK, D_H), lambda i: (i, 0)),
            pl.BlockSpec((D_H,), lambda i: (0,)),
            pl.BlockSpec((D_H, N_OUT), lambda i: (0, 0)),
            pl.BlockSpec((N_OUT,), lambda i: (0,)),
        ],
        out_specs=pl.BlockSpec((ROW_BLK, N_OUT), lambda i: (i, 0)),
        out_shape=jax.ShapeDtypeStruct((N_NODES, N_OUT), jnp.float32),
    )(aggdeg, self_msg, b, lin_w, lin_b)


def _edge_phase(y, pq, edges):
    src, dst = edges[0], edges[1]
    mask = (src != dst).astype(jnp.float32)
    ps = pq[src, :HEADS]
    pd = pq[dst, HEADS:]
    q = jax.nn.softmax(ps - pd, axis=1) * mask[:, None]
    yj = y[src].reshape(-1, HEADS, D_H)
    msg = (yj * q[:, :, None]).sum(axis=1)
    agg = jax.ops.segment_sum(msg, dst, num_segments=N_NODES)
    deg = jax.ops.segment_sum(mask, dst, num_segments=N_NODES)
    blk = jnp.concatenate(
        [agg, deg[:, None], jnp.zeros((N_NODES, 15), jnp.float32)], axis=1)
    return jnp.stack([blk, jnp.zeros_like(blk)], axis=0)


def _prep_layer(W, U, c):
    Upad = jnp.concatenate([U, U], axis=1)
    cvec = jnp.concatenate([jnp.zeros((HEADS,), jnp.float32), c])
    qs = jax.nn.softmax(c)
    Wself = (W.reshape(D_H, HEADS, D_H) * qs[None, :, None]).sum(axis=1)
    return Upad, cvec, Wself


def kernel(verts, edges, fc0_w, fc0_b, conv1_W, conv1_U, conv1_c, conv1_b,
           conv2_W, conv2_U, conv2_c, conv2_b, lin_w, lin_b):
    Upad1, cvec1, Wself1 = _prep_layer(conv1_W, conv1_U, conv1_c)
    Upad2, cvec2, Wself2 = _prep_layer(conv2_W, conv2_U, conv2_c)

    y1, pq1, self1 = _prologue1(verts, fc0_w, fc0_b, conv1_W, Upad1, cvec1,
                                Wself1)
    aggdeg1 = _edge_phase(y1, pq1, edges)
    y2, pq2, self2 = _mid(aggdeg1, self1, conv1_b, conv2_W, Upad2, cvec2,
                          Wself2)
    aggdeg2 = _edge_phase(y2, pq2, edges)
    return _final(aggdeg2, self2, conv2_b, lin_w, lin_b)
